# Initial kernel scaffold; baseline (speedup 1.0000x reference)
import jax
import jax.numpy as jnp
from jax.experimental import pallas as pl


def kernel(x, emb, W1, b1, W2, b2, W3, b3):
    raise NotImplementedError("write your pallas kernel here")


# speedup vs baseline: 1.3118x; 1.3118x over previous
import functools

import jax
import jax.numpy as jnp
from jax import lax
from jax.experimental import pallas as pl
from jax.experimental.pallas import tpu as pltpu
from jax.experimental.pallas import tpu_sc as plsc

B = 16384
L = 200
HID = 64
H2 = 2 * HID
NCLS = 100
NP = 128

NC = 2
NS = 16
NW = NC * NS
SEGS_W = B // NW
PER_W = SEGS_W * L
K = 128
NCH = PER_W // K


def _pool_body(x_ref, seg_ref, emb_ref, zeros_ref, out_ref,
               acc, idxb, segb, rows, sem):
    c = lax.axis_index("c")
    s = lax.axis_index("s")
    wid = s * NC + c
    pltpu.sync_copy(zeros_ref, acc.at[pl.ds(s * SEGS_W, SEGS_W)])

    def body(i, carry):
        pltpu.sync_copy(x_ref.at[wid, i], idxb)
        pltpu.sync_copy(seg_ref.at[s, i], segb)
        pltpu.async_copy(emb_ref.at[idxb], rows, sem).wait()
        pltpu.sync_copy(rows, acc.at[segb], add=True)
        return carry

    lax.fori_loop(0, NCH, body, 0)
    pltpu.sync_copy(acc.at[pl.ds(s * SEGS_W, SEGS_W)],
                    out_ref.at[pl.ds(wid * SEGS_W, SEGS_W)])


@jax.jit
def _pool(x_r, seg_ids, emb, zeros):
    mesh = plsc.VectorSubcoreMesh(core_axis_name="c", subcore_axis_name="s")
    f = functools.partial(
        pl.kernel,
        out_type=jax.ShapeDtypeStruct((B, HID), jnp.float32),
        mesh=mesh,
        scratch_types=[
            pltpu.VMEM_SHARED((NS * SEGS_W, HID), jnp.float32),
            pltpu.VMEM((K,), jnp.int32),
            pltpu.VMEM((K,), jnp.int32),
            pltpu.VMEM((K, HID), jnp.float32),
            pltpu.SemaphoreType.DMA,
        ],
        compiler_params=pltpu.CompilerParams(use_tc_tiling_on_sc=False),
    )(_pool_body)
    return f(x_r, seg_ids, emb, zeros)


def _mlp_body(p_ref, w1_ref, b1_ref, w2_ref, b2_ref, w3_ref, b3_ref, o_ref):
    h = jnp.dot(p_ref[...], w1_ref[...], preferred_element_type=jnp.float32)
    h = jnp.maximum(h + b1_ref[...], 0.0)
    h = jnp.dot(h, w2_ref[...], preferred_element_type=jnp.float32)
    h = jnp.maximum(h + b2_ref[...], 0.0)
    o_ref[...] = jnp.dot(h, w3_ref[...],
                         preferred_element_type=jnp.float32) + b3_ref[...]


@jax.jit
def _mlp(pooled, w1, b1, w2, b2, w3, b3):
    BM = 2048
    grid = (B // BM,)
    return pl.pallas_call(
        _mlp_body,
        grid=grid,
        in_specs=[
            pl.BlockSpec((BM, HID), lambda i: (i, 0)),
            pl.BlockSpec((HID, H2), lambda i: (0, 0)),
            pl.BlockSpec((1, H2), lambda i: (0, 0)),
            pl.BlockSpec((H2, H2), lambda i: (0, 0)),
            pl.BlockSpec((1, H2), lambda i: (0, 0)),
            pl.BlockSpec((H2, NP), lambda i: (0, 0)),
            pl.BlockSpec((1, NP), lambda i: (0, 0)),
        ],
        out_specs=pl.BlockSpec((BM, NP), lambda i: (i, 0)),
        out_shape=jax.ShapeDtypeStruct((B, NP), jnp.float32),
    )(pooled, w1, b1, w2, b2, w3, b3)


def kernel(x, emb, W1, b1, W2, b2, W3, b3):
    x_r = x.astype(jnp.int32).reshape(NW, NCH, K)
    local = (jnp.arange(PER_W, dtype=jnp.int32) // L).reshape(1, NCH, K)
    offs = (jnp.arange(NS, dtype=jnp.int32) * SEGS_W).reshape(NS, 1, 1)
    seg_ids = local + offs
    zeros = jnp.zeros((SEGS_W, HID), jnp.float32)
    pooled = _pool(x_r, seg_ids, emb, zeros)

    w1 = W1 * (1.0 / L)
    w3 = jnp.pad(W3, ((0, 0), (0, NP - NCLS)))
    b3p = jnp.pad(b3, (0, NP - NCLS))
    out = _mlp(pooled, w1, b1.reshape(1, H2), W2, b2.reshape(1, H2),
               w3, b3p.reshape(1, NP))
    return out[:, :NCLS]


# speedup vs baseline: 3.4774x; 2.6508x over previous
import functools

import jax
import jax.numpy as jnp
from jax import lax
from jax.experimental import pallas as pl
from jax.experimental.pallas import tpu as pltpu
from jax.experimental.pallas import tpu_sc as plsc

B = 16384
L = 200
HID = 64
H2 = 2 * HID
NCLS = 100
NP = 128

NC = 2
NS = 16
NW = NC * NS
BLK = 128
NBLK = B // BLK
BLKS_W = NBLK // NW
KQ = 8


def _pool_body(xt_ref, emb_ref, out_ref, idxb, dst, sem):
    c = lax.axis_index("c")
    s = lax.axis_index("s")
    wid = s * NC + c

    def block(k, carry):
        bb = wid * BLKS_W + k
        pltpu.sync_copy(xt_ref.at[bb], idxb)
        pltpu.async_copy(emb_ref.at[idxb.at[0]], dst, sem).wait()
        for t in range(1, 1 + KQ):
            pltpu.async_copy(emb_ref.at[idxb.at[t]], dst, sem, add=True)

        def tok(t, carry2):
            pltpu.async_copy(emb_ref.at[idxb.at[t]], dst, sem, add=True)
            pltpu.make_async_copy(emb_ref.at[idxb.at[0]], dst, sem).wait()
            return carry2

        lax.fori_loop(1 + KQ, L, tok, 0)
        for _ in range(KQ):
            pltpu.make_async_copy(emb_ref.at[idxb.at[0]], dst, sem).wait()
        pltpu.sync_copy(dst, out_ref.at[pl.ds(bb * BLK, BLK)])
        return carry

    lax.fori_loop(0, BLKS_W, block, 0)


@jax.jit
def _pool(x_t, emb):
    mesh = plsc.VectorSubcoreMesh(core_axis_name="c", subcore_axis_name="s")
    f = functools.partial(
        pl.kernel,
        out_type=jax.ShapeDtypeStruct((B, HID), jnp.float32),
        mesh=mesh,
        scratch_types=[
            pltpu.VMEM((L, BLK), jnp.int32),
            pltpu.VMEM((BLK, HID), jnp.float32),
            pltpu.SemaphoreType.DMA,
        ],
        compiler_params=pltpu.CompilerParams(use_tc_tiling_on_sc=False),
    )(_pool_body)
    return f(x_t, emb)


def _mlp_body(p_ref, w1_ref, b1_ref, w2_ref, b2_ref, w3_ref, b3_ref, o_ref):
    h = jnp.dot(p_ref[...], w1_ref[...], preferred_element_type=jnp.float32)
    h = jnp.maximum(h + b1_ref[...], 0.0)
    h = jnp.dot(h, w2_ref[...], preferred_element_type=jnp.float32)
    h = jnp.maximum(h + b2_ref[...], 0.0)
    o_ref[...] = jnp.dot(h, w3_ref[...],
                         preferred_element_type=jnp.float32) + b3_ref[...]


@jax.jit
def _mlp(pooled, w1, b1, w2, b2, w3, b3):
    BM = 2048
    grid = (B // BM,)
    return pl.pallas_call(
        _mlp_body,
        grid=grid,
        in_specs=[
            pl.BlockSpec((BM, HID), lambda i: (i, 0)),
            pl.BlockSpec((HID, H2), lambda i: (0, 0)),
            pl.BlockSpec((1, H2), lambda i: (0, 0)),
            pl.BlockSpec((H2, H2), lambda i: (0, 0)),
            pl.BlockSpec((1, H2), lambda i: (0, 0)),
            pl.BlockSpec((H2, NP), lambda i: (0, 0)),
            pl.BlockSpec((1, NP), lambda i: (0, 0)),
        ],
        out_specs=pl.BlockSpec((BM, NP), lambda i: (i, 0)),
        out_shape=jax.ShapeDtypeStruct((B, NP), jnp.float32),
    )(pooled, w1, b1, w2, b2, w3, b3)


def kernel(x, emb, W1, b1, W2, b2, W3, b3):
    x_t = x.astype(jnp.int32).reshape(NBLK, BLK, L).transpose(0, 2, 1)
    pooled = _pool(x_t, emb)

    w1 = W1 * (1.0 / L)
    w3 = jnp.pad(W3, ((0, 0), (0, NP - NCLS)))
    b3p = jnp.pad(b3, (0, NP - NCLS))
    out = _mlp(pooled, w1, b1.reshape(1, H2), W2, b2.reshape(1, H2),
               w3, b3p.reshape(1, NP))
    return out[:, :NCLS]
